# Initial kernel scaffold; baseline (speedup 1.0000x reference)
import jax
import jax.numpy as jnp
from jax.experimental import pallas as pl


def kernel(items_emb, item_grad_bank, items, items_emb_grad):
    raise NotImplementedError("write your pallas kernel here")


# speedup vs baseline: 1.1935x; 1.1935x over previous
import jax
import jax.numpy as jnp
from jax.experimental import pallas as pl
from jax.experimental.pallas import tpu as pltpu
from jax.experimental.pallas import tpu_sc as plsc

M = 100000
D = 32
B = 16384
K = 1000
LR = 0.01
LIMIT = 1.0

NC = 2
NS = 16
HALF = M // NC
RPT = HALF // NS
GPT = B // NS
NCHUNK = GPT // 128

ROW_BLK = 1024
NBLK = (M + ROW_BLK - 1) // ROW_BLK
NPAD_ROWS = NBLK * ROW_BLK // 128


def _clip_body(x_ref, o_ref):
    x = x_ref[...]
    n = jnp.sqrt(jnp.sum(x * x, axis=1, keepdims=True))
    scale = jnp.where(n > LIMIT, LIMIT / jnp.maximum(n, 1e-12), 1.0)
    o_ref[...] = x * scale


def _sc_bank_body(bank_in, g3d, items3d, bank_out, acc, g_buf, idx_buf):
    c = jax.lax.axis_index("c")
    s = jax.lax.axis_index("s")
    base_row = c * HALF + s * RPT
    pltpu.sync_copy(bank_in.at[pl.ds(base_row, RPT)],
                    acc.at[pl.ds(s * RPT, RPT)])
    pltpu.sync_copy(items3d.at[s], idx_buf)
    lo = c * HALF
    for j in range(NCHUNK):
        for q in range(128 // 16):
            v = idx_buf[j, pl.ds(q * 16, 16)]
            local = v - lo
            oob = (local < 0) | (local >= HALF)
            idx_buf[j, pl.ds(q * 16, 16)] = jnp.where(oob, HALF, local)
    plsc.subcore_barrier()
    for h in range(2):
        pltpu.sync_copy(g3d.at[s, h], g_buf)
        for j in range(NCHUNK // 2):
            pltpu.sync_copy(g_buf.at[j],
                            acc.at[idx_buf.at[h * (NCHUNK // 2) + j]],
                            add=True)
    plsc.subcore_barrier()
    pltpu.sync_copy(acc.at[pl.ds(s * RPT, RPT)],
                    bank_out.at[pl.ds(base_row, RPT)])


def _norm_body(b_ref, o_ref):
    i = pl.program_id(0)
    x = b_ref[...]
    x3 = x.reshape(8, 128, D)
    n2 = jnp.sum(x3 * x3, axis=2)
    rid = (i * ROW_BLK
           + jax.lax.broadcasted_iota(jnp.int32, (8, 128), 0) * 128
           + jax.lax.broadcasted_iota(jnp.int32, (8, 128), 1))
    o_ref[...] = jnp.where(rid < M, n2, 0.0)


def _thresh_body(n2_ref, v_ref, t_ref):
    x = n2_ref[...]
    bits = jax.lax.bitcast_convert_type(x, jnp.int32)

    def step(i, u):
        b = 30 - i
        cand = jnp.bitwise_or(u, jnp.left_shift(jnp.int32(1), b))
        cnt = jnp.sum((bits >= cand).astype(jnp.int32))
        return jnp.where(cnt >= K, cand, u)

    u = jax.lax.fori_loop(0, 31, step, jnp.int32(0))
    v = jax.lax.bitcast_convert_type(u, jnp.float32)
    sel = x >= v
    tsum = jnp.sum(jnp.where(sel, jnp.sqrt(x), 0.0))
    v_ref[...] = jnp.full((1, 1), v, jnp.float32)
    t_ref[...] = jnp.full((1, 1), tsum / K, jnp.float32)


def _update_body(e_ref, b_ref, v_ref, t_ref, o_ref):
    v = v_ref[0, 0]
    t = t_ref[0, 0]
    e = e_ref[...]
    bk = b_ref[...]
    n2 = jnp.sum(bk * bk, axis=1, keepdims=True)
    bn = jnp.sqrt(n2)
    scale = jnp.where(bn > t, t / jnp.maximum(bn, 1e-12), 1.0)
    upd = jnp.where(n2 >= v, bk * scale, 0.0)
    o_ref[...] = e - LR * upd


def kernel(items_emb, item_grad_bank, items, items_emb_grad):
    f32 = jnp.float32
    g = pl.pallas_call(
        _clip_body,
        grid=(B // ROW_BLK,),
        in_specs=[pl.BlockSpec((ROW_BLK, D), lambda i: (i, 0))],
        out_specs=pl.BlockSpec((ROW_BLK, D), lambda i: (i, 0)),
        out_shape=jax.ShapeDtypeStruct((B, D), f32),
    )(items_emb_grad)

    g3d = g.reshape(NS, 2, NCHUNK // 2, 128, D)
    items3d = items.reshape(NS, NCHUNK, 128)
    mesh = plsc.VectorSubcoreMesh(core_axis_name="c", subcore_axis_name="s")
    bank = pl.kernel(
        _sc_bank_body,
        out_type=jax.ShapeDtypeStruct((M, D), f32),
        mesh=mesh,
        scratch_types=[
            pltpu.VMEM_SHARED((HALF + 8, D), f32),
            pltpu.VMEM((NCHUNK // 2, 128, D), f32),
            pltpu.VMEM((NCHUNK, 128), jnp.int32),
        ],
        compiler_params=pltpu.CompilerParams(use_tc_tiling_on_sc=False),
    )(item_grad_bank, g3d, items3d)

    n2p = pl.pallas_call(
        _norm_body,
        grid=(NBLK,),
        in_specs=[pl.BlockSpec((ROW_BLK, D), lambda i: (i, 0))],
        out_specs=pl.BlockSpec((8, 128), lambda i: (i, 0)),
        out_shape=jax.ShapeDtypeStruct((NPAD_ROWS, 128), f32),
    )(bank)

    v, t = pl.pallas_call(
        _thresh_body,
        in_specs=[pl.BlockSpec((NPAD_ROWS, 128), lambda: (0, 0))],
        out_specs=[pl.BlockSpec((1, 1), lambda: (0, 0)),
                   pl.BlockSpec((1, 1), lambda: (0, 0))],
        out_shape=[jax.ShapeDtypeStruct((1, 1), f32),
                   jax.ShapeDtypeStruct((1, 1), f32)],
    )(n2p)

    out = pl.pallas_call(
        _update_body,
        grid=(NBLK,),
        in_specs=[
            pl.BlockSpec((ROW_BLK, D), lambda i: (i, 0)),
            pl.BlockSpec((ROW_BLK, D), lambda i: (i, 0)),
            pl.BlockSpec(memory_space=pltpu.MemorySpace.SMEM),
            pl.BlockSpec(memory_space=pltpu.MemorySpace.SMEM),
        ],
        out_specs=pl.BlockSpec((ROW_BLK, D), lambda i: (i, 0)),
        out_shape=jax.ShapeDtypeStruct((M, D), f32),
    )(items_emb, bank, v, t)
    return out
